# baseline (speedup 1.0000x reference)
import functools

import jax
import jax.numpy as jnp
from jax import lax
from jax.experimental import pallas as pl
from jax.experimental.pallas import tpu as pltpu
from jax.experimental.pallas import tpu_sc as plsc

VOCAB = 1000000
EMBED = 64
BATCH = 4096
SEQ = 200
NC = 2
NS = 16
NW = NC * NS
BW = BATCH // NW
RB = 4
LOOKAHEAD = RB - 1

_mesh = plsc.VectorSubcoreMesh(core_axis_name="c", subcore_axis_name="s")


@functools.partial(
    pl.kernel,
    mesh=_mesh,
    out_type=jax.ShapeDtypeStruct((BATCH, SEQ, EMBED), jnp.float32),
    scratch_types=[
        pltpu.VMEM((SEQ, BW), jnp.int32),
        pltpu.VMEM((RB, BW, EMBED), jnp.float32),
        pltpu.SemaphoreType.DMA((RB,)),
        pltpu.SemaphoreType.DMA((RB,)),
    ],
    compiler_params=pltpu.CompilerParams(
        needs_layout_passes=False, use_tc_tiling_on_sc=False
    ),
)
def _embed(xt_hbm, table_hbm, out_hbm, idx_v, rows_v, gsem, wsem):
    wid = lax.axis_index("s") * NC + lax.axis_index("c")
    b0 = wid * BW

    zeros16 = jnp.zeros((16,), jnp.float32)
    lane = lax.iota(jnp.int32, 16)

    pltpu.sync_copy(xt_hbm.at[:, pl.ds(b0, BW)], idx_v)

    def fire_gather(u, r):
        pltpu.async_copy(table_hbm.at[idx_v.at[u]], rows_v.at[r], gsem.at[r])

    def wait_gather(u, r):
        pltpu.make_async_copy(
            table_hbm.at[idx_v.at[u]], rows_v.at[r], gsem.at[r]
        ).wait()

    def fire_write(u, r):
        pltpu.async_copy(
            rows_v.at[r],
            out_hbm.at[pl.ds(b0, BW), u, :],
            wsem.at[r],
        )

    def wait_write(u, r):
        pltpu.make_async_copy(
            rows_v.at[r],
            out_hbm.at[pl.ds(b0, BW), u, :],
            wsem.at[r],
        ).wait()

    def fix(u, r):
        def fix_body(i, fcarry):
            idxv = idx_v[u, pl.ds(i * 16, 16)]
            m = idxv == 0
            nzero = plsc.all_reduce_population_count(m)

            @pl.when(nzero[0] > 0)
            def _zero_rows():
                rows16 = i * 16 + lane
                for c in range(EMBED):
                    plsc.store_scatter(
                        rows_v.at[r],
                        [rows16, jnp.full((16,), c, jnp.int32)],
                        zeros16,
                        mask=m,
                    )

            return fcarry

        lax.fori_loop(0, BW // 16, fix_body, 0)

    for r in range(LOOKAHEAD):
        fire_gather(r, r)

    def block_body(p, carry):
        for r in range(RB):
            u = p * RB + r
            ra = (r + LOOKAHEAD) % RB
            ua = u + LOOKAHEAD

            @pl.when(ua < SEQ)
            def _ahead():
                @pl.when(ua >= RB)
                def _reuse_wait():
                    wait_write(ua - RB, ra)

                fire_gather(ua, ra)

            @pl.when(u < SEQ)
            def _iter():
                wait_gather(u, r)
                fix(u, r)
                fire_write(u, r)

        return carry

    lax.fori_loop(0, SEQ // RB, block_body, 0)

    for r in range(RB):
        wait_write(0, r)


def kernel(x, E):
    xt = x.astype(jnp.int32).T
    return _embed(xt, E)
